# Initial kernel scaffold; baseline (speedup 1.0000x reference)
import jax
import jax.numpy as jnp
from jax.experimental import pallas as pl


def kernel(x, edge_index, W1, b1, W2, b2, Wh, bh):
    raise NotImplementedError("write your pallas kernel here")


# speedup vs baseline: 37.7557x; 37.7557x over previous
import functools
import jax
import jax.numpy as jnp
from jax import lax
from jax.experimental import pallas as pl
from jax.experimental.pallas import tpu as pltpu
from jax.experimental.pallas import tpu_sc as plsc

NNODE = 10000
NEDGE = 160000
NKEY = NNODE * NNODE
NC, NS, LN = 2, 16, 16
NW = NC * NS
NPAD = 10240
TRASH = 10000
EW = NEDGE // NW
ET = NEDGE // NS
CH = 128
BN = 400
NB = NNODE // BN

_mesh = plsc.VectorSubcoreMesh(core_axis_name="c", subcore_axis_name="s")


def _iota16():
    return lax.iota(jnp.int32, 16)


def _a1_body(src_hbm, dst_hbm, table_hbm, src_v, dst_v, key_v, eid_v,
             src_t, dst_t, key_t, eid_t, sem):
    c = lax.axis_index("c")
    s = lax.axis_index("s")
    wid = c * NS + s
    base = wid * EW
    iota = _iota16()

    @pl.loop(0, 39)
    def _chunks(j):
        off = base + j * CH
        pltpu.sync_copy(src_hbm.at[pl.ds(off, CH)], src_v)
        pltpu.sync_copy(dst_hbm.at[pl.ds(off, CH)], dst_v)
        for i in range(CH // 16):
            sl = pl.ds(i * 16, 16)
            key_v[sl] = src_v[sl] * NNODE + dst_v[sl]
            eid_v[sl] = off + (i * 16) + iota
        pltpu.async_copy(eid_v, table_hbm.at[key_v], sem).wait()

    off = base + (EW - 16)
    pltpu.sync_copy(src_hbm.at[pl.ds(off, 16)], src_t)
    pltpu.sync_copy(dst_hbm.at[pl.ds(off, 16)], dst_t)
    key_t[...] = src_t[...] * NNODE + dst_t[...]
    eid_t[...] = off + iota
    pltpu.async_copy(eid_t, table_hbm.at[key_t], sem).wait()


@jax.jit
def _scatter_ids(src, dst):
    return pl.kernel(
        _a1_body,
        out_type=jax.ShapeDtypeStruct((NKEY,), jnp.int32),
        mesh=_mesh,
        scratch_types=[
            pltpu.VMEM((CH,), jnp.int32),
            pltpu.VMEM((CH,), jnp.int32),
            pltpu.VMEM((CH,), jnp.int32),
            pltpu.VMEM((CH,), jnp.int32),
            pltpu.VMEM((16,), jnp.int32),
            pltpu.VMEM((16,), jnp.int32),
            pltpu.VMEM((16,), jnp.int32),
            pltpu.VMEM((16,), jnp.int32),
            pltpu.SemaphoreType.DMA,
        ],
    )(src, dst)


def _a2_body(src_hbm, dst_hbm, table_hbm, eff_hbm, deg_hbm,
             src_v, dst_v, key_v, win_v, eff_v, val_v,
             src_t, dst_t, key_t, win_t, eff_t, val_t,
             zbuf, deg_sp, sem):
    c = lax.axis_index("c")
    s = lax.axis_index("s")
    wid = c * NS + s
    base = wid * EW
    iota = _iota16()
    zeros_f = jnp.zeros((16,), jnp.float32)

    zbuf[...] = zeros_f

    @pl.loop(0, NPAD // NS // 16)
    def _zd(k):
        row = s * (NPAD // NS) + k * 16
        pltpu.sync_copy(zbuf, deg_sp.at[pl.ds(row, 16)])

    plsc.subcore_barrier()

    @pl.loop(0, 39)
    def _chunks(j):
        off = base + j * CH
        pltpu.sync_copy(src_hbm.at[pl.ds(off, CH)], src_v)
        pltpu.sync_copy(dst_hbm.at[pl.ds(off, CH)], dst_v)
        for i in range(CH // 16):
            sl = pl.ds(i * 16, 16)
            key_v[sl] = src_v[sl] * NNODE + dst_v[sl]
        pltpu.async_copy(table_hbm.at[key_v], win_v, sem).wait()
        for i in range(CH // 16):
            sl = pl.ds(i * 16, 16)
            eid = off + (i * 16) + iota
            flag = win_v[sl] == eid
            eff_v[sl] = jnp.where(flag, dst_v[sl], TRASH)
            val_v[sl] = jnp.where(flag, 1.0, 0.0).astype(jnp.float32)
        pltpu.sync_copy(eff_v, eff_hbm.at[pl.ds(off, CH)])
        pltpu.sync_copy(val_v, deg_sp.at[dst_v], add=True)

    off = base + (EW - 16)
    pltpu.sync_copy(src_hbm.at[pl.ds(off, 16)], src_t)
    pltpu.sync_copy(dst_hbm.at[pl.ds(off, 16)], dst_t)
    key_t[...] = src_t[...] * NNODE + dst_t[...]
    pltpu.async_copy(table_hbm.at[key_t], win_t, sem).wait()
    eid = off + iota
    flag = win_t[...] == eid
    eff_t[...] = jnp.where(flag, dst_t[...], TRASH)
    new = iota >= 8
    val_t[...] = jnp.where(flag & new, 1.0, 0.0).astype(jnp.float32)
    pltpu.sync_copy(eff_t, eff_hbm.at[pl.ds(off, 16)])
    pltpu.sync_copy(val_t, deg_sp.at[dst_t], add=True)

    plsc.subcore_barrier()

    stripe = NPAD // NS
    row = s * stripe
    pltpu.sync_copy(deg_sp.at[pl.ds(row, stripe)],
                    deg_hbm.at[pl.ds(c * NPAD + row, stripe)])


@jax.jit
def _flags_deg(src, dst, table):
    return pl.kernel(
        _a2_body,
        out_type=(
            jax.ShapeDtypeStruct((NEDGE,), jnp.int32),
            jax.ShapeDtypeStruct((2 * NPAD,), jnp.float32),
        ),
        mesh=_mesh,
        scratch_types=[
            pltpu.VMEM((CH,), jnp.int32),
            pltpu.VMEM((CH,), jnp.int32),
            pltpu.VMEM((CH,), jnp.int32),
            pltpu.VMEM((CH,), jnp.int32),
            pltpu.VMEM((CH,), jnp.int32),
            pltpu.VMEM((CH,), jnp.float32),
            pltpu.VMEM((16,), jnp.int32),
            pltpu.VMEM((16,), jnp.int32),
            pltpu.VMEM((16,), jnp.int32),
            pltpu.VMEM((16,), jnp.int32),
            pltpu.VMEM((16,), jnp.int32),
            pltpu.VMEM((16,), jnp.float32),
            pltpu.VMEM((16,), jnp.float32),
            pltpu.VMEM_SHARED((NPAD,), jnp.float32),
            pltpu.SemaphoreType.DMA,
        ],
    )(src, dst, table)


def _msg_body(zs_hbm, src_hbm, eff_hbm, p_hbm,
              src_v, idx_v, dst_v, rows_v,
              src_t, idx_t, dst_t, rows_t,
              zbuf, acc_sp, sem):
    c = lax.axis_index("c")
    s = lax.axis_index("s")
    base = s * ET
    coff = c * NNODE
    zeros_f = jnp.zeros((16,), jnp.float32)

    @pl.loop(0, 16)
    def _zb(r):
        for k in range(8):
            zbuf[r, pl.ds(k * 16, 16)] = zeros_f

    @pl.loop(0, NPAD // NS // 16)
    def _za(k):
        row = s * (NPAD // NS) + k * 16
        pltpu.sync_copy(zbuf, acc_sp.at[pl.ds(row, 16)])

    plsc.subcore_barrier()

    @pl.loop(0, 78)
    def _chunks(j):
        off = base + j * CH
        pltpu.sync_copy(src_hbm.at[pl.ds(off, CH)], src_v)
        pltpu.sync_copy(eff_hbm.at[pl.ds(off, CH)], dst_v)
        for i in range(CH // 16):
            sl = pl.ds(i * 16, 16)
            idx_v[sl] = src_v[sl] + coff
        pltpu.async_copy(zs_hbm.at[idx_v], rows_v, sem).wait()
        pltpu.sync_copy(rows_v, acc_sp.at[dst_v], add=True)

    off = base + (ET - 16)
    pltpu.sync_copy(src_hbm.at[pl.ds(off, 16)], src_t)
    pltpu.sync_copy(eff_hbm.at[pl.ds(off, 16)], dst_t)
    idx_t[...] = src_t[...] + coff
    pltpu.async_copy(zs_hbm.at[idx_t], rows_t, sem).wait()
    pltpu.sync_copy(rows_t, acc_sp.at[dst_t], add=True)

    plsc.subcore_barrier()

    stripe = NPAD // NS
    row = s * stripe
    pltpu.sync_copy(acc_sp.at[pl.ds(row, stripe)],
                    p_hbm.at[pl.ds(c * NPAD + row, stripe)])


@jax.jit
def _msg(zs, src, eff):
    return pl.kernel(
        _msg_body,
        out_type=jax.ShapeDtypeStruct((2 * NPAD, 128), jnp.float32),
        mesh=_mesh,
        scratch_types=[
            pltpu.VMEM((CH,), jnp.int32),
            pltpu.VMEM((CH,), jnp.int32),
            pltpu.VMEM((CH,), jnp.int32),
            pltpu.VMEM((CH, 128), jnp.float32),
            pltpu.VMEM((16,), jnp.int32),
            pltpu.VMEM((16,), jnp.int32),
            pltpu.VMEM((16,), jnp.int32),
            pltpu.VMEM((16, 128), jnp.float32),
            pltpu.VMEM((16, 128), jnp.float32),
            pltpu.VMEM_SHARED((NPAD, 128), jnp.float32),
            pltpu.SemaphoreType.DMA,
        ],
    )(zs, src, eff)


def _dinv_block(deg0_ref, deg1_ref):
    deg = deg0_ref[...] + deg1_ref[...] + 1.0
    return lax.rsqrt(deg)


def _bias_row(b_ref, j):
    rows = lax.broadcasted_iota(jnp.int32, (2, 128), 0)
    return jnp.sum(jnp.where(rows == j, b_ref[...], 0.0), axis=0,
                   keepdims=True)


def _tc1_body(x_ref, w_ref, b_ref, deg0_ref, deg1_ref, o_ref):
    j = pl.program_id(1)
    dinv = _dinv_block(deg0_ref, deg1_ref)
    b = _bias_row(b_ref, j)
    z = jnp.dot(x_ref[...], w_ref[...], preferred_element_type=jnp.float32)
    o_ref[...] = dinv * (z + b)


@jax.jit
def _tc1(x, W1, b1, deg0, deg1):
    b1r = b1.reshape(2, 128)
    return pl.pallas_call(
        _tc1_body,
        grid=(NB, 2),
        in_specs=[
            pl.BlockSpec((BN, 256), lambda i, j: (i, 0)),
            pl.BlockSpec((256, 128), lambda i, j: (0, j)),
            pl.BlockSpec((2, 128), lambda i, j: (0, 0)),
            pl.BlockSpec((BN, 1), lambda i, j: (i, 0)),
            pl.BlockSpec((BN, 1), lambda i, j: (i, 0)),
        ],
        out_specs=pl.BlockSpec((BN, 128), lambda i, j: (j * NB + i, 0)),
        out_shape=jax.ShapeDtypeStruct((2 * NNODE, 128), jnp.float32),
    )(x, W1, b1r, deg0, deg1)


def _tc2_body(pa_ref, pb_ref, za_ref, zb_ref, w_ref, b_ref,
              deg0_ref, deg1_ref, o_ref):
    j = pl.program_id(1)
    dinv = _dinv_block(deg0_ref, deg1_ref)
    ha = jnp.maximum(dinv * (pa_ref[...] + za_ref[...]), 0.0)
    hb = jnp.maximum(dinv * (pb_ref[...] + zb_ref[...]), 0.0)
    h = jnp.concatenate([ha, hb], axis=1)
    b = _bias_row(b_ref, j)
    z = jnp.dot(h, w_ref[...], preferred_element_type=jnp.float32)
    o_ref[...] = dinv * (z + b)


@jax.jit
def _tc2(pa, pb, zs, W2, b2, deg0, deg1):
    b2r = b2.reshape(2, 128)
    return pl.pallas_call(
        _tc2_body,
        grid=(NB, 2),
        in_specs=[
            pl.BlockSpec((BN, 128), lambda i, j: (i, 0)),
            pl.BlockSpec((BN, 128), lambda i, j: (i, 0)),
            pl.BlockSpec((BN, 128), lambda i, j: (i, 0)),
            pl.BlockSpec((BN, 128), lambda i, j: (NB + i, 0)),
            pl.BlockSpec((256, 128), lambda i, j: (0, j)),
            pl.BlockSpec((2, 128), lambda i, j: (0, 0)),
            pl.BlockSpec((BN, 1), lambda i, j: (i, 0)),
            pl.BlockSpec((BN, 1), lambda i, j: (i, 0)),
        ],
        out_specs=pl.BlockSpec((BN, 128), lambda i, j: (j * NB + i, 0)),
        out_shape=jax.ShapeDtypeStruct((2 * NNODE, 128), jnp.float32),
    )(pa, pb, zs, zs, W2, b2r, deg0, deg1)


def _tc3_body(pa_ref, pb_ref, za_ref, zb_ref, w_ref, b_ref,
              deg0_ref, deg1_ref, o_ref):
    dinv = _dinv_block(deg0_ref, deg1_ref)
    ha = jnp.maximum(dinv * (pa_ref[...] + za_ref[...]), 0.0)
    hb = jnp.maximum(dinv * (pb_ref[...] + zb_ref[...]), 0.0)
    h = jnp.concatenate([ha, hb], axis=1)
    logits = jnp.dot(h, w_ref[...], preferred_element_type=jnp.float32)
    logits = logits + b_ref[...]
    m = jnp.max(logits, axis=1, keepdims=True)
    lse = m + jnp.log(jnp.sum(jnp.exp(logits - m), axis=1, keepdims=True))
    o_ref[...] = logits - lse


@jax.jit
def _tc3(pa, pb, zs, Wh, bh, deg0, deg1):
    bhr = bh.reshape(1, 2)
    return pl.pallas_call(
        _tc3_body,
        grid=(NB,),
        in_specs=[
            pl.BlockSpec((BN, 128), lambda i: (i, 0)),
            pl.BlockSpec((BN, 128), lambda i: (i, 0)),
            pl.BlockSpec((BN, 128), lambda i: (i, 0)),
            pl.BlockSpec((BN, 128), lambda i: (NB + i, 0)),
            pl.BlockSpec((256, 2), lambda i: (0, 0)),
            pl.BlockSpec((1, 2), lambda i: (0, 0)),
            pl.BlockSpec((BN, 1), lambda i: (i, 0)),
            pl.BlockSpec((BN, 1), lambda i: (i, 0)),
        ],
        out_specs=pl.BlockSpec((BN, 2), lambda i: (i, 0)),
        out_shape=jax.ShapeDtypeStruct((NNODE, 2), jnp.float32),
    )(pa, pb, zs, zs, Wh, bhr, deg0, deg1)


def kernel(x, edge_index, W1, b1, W2, b2, Wh, bh):
    src = edge_index[0]
    dst = edge_index[1]
    table = _scatter_ids(src, dst)
    eff, deg = _flags_deg(src, dst, table)
    deg0 = deg[:NNODE].reshape(NNODE, 1)
    deg1 = deg[NPAD:NPAD + NNODE].reshape(NNODE, 1)
    zs1 = _tc1(x, W1, b1, deg0, deg1)
    p1 = _msg(zs1, src, eff)
    zs2 = _tc2(p1[:NNODE], p1[NPAD:NPAD + NNODE], zs1, W2, b2, deg0, deg1)
    p2 = _msg(zs2, src, eff)
    return _tc3(p2[:NNODE], p2[NPAD:NPAD + NNODE], zs2, Wh, bh, deg0, deg1)
